# Initial kernel scaffold; baseline (speedup 1.0000x reference)
import functools

import jax
import jax.numpy as jnp
from jax import lax
from jax.experimental import pallas as pl
from jax.experimental.pallas import tpu as pltpu
from jax.experimental.pallas import tpu_sc as plsc

VOCAB = 1000000
D = 64
B = 16384
L = 50
NUM_CLASS = 5

NC = 2
NS = 16
NW = NC * NS
BAGS_PER_W = B // NW
BAGS_PER_CHUNK = 2
ROWS_PER_CHUNK = BAGS_PER_CHUNK * L
NCHUNK = BAGS_PER_W // BAGS_PER_CHUNK
NBUF = 2


def _sc_body(text_hbm, table_hbm, out_hbm, idx_v, gbuf, outb, sem0, sem1):
    wid = lax.axis_index("s") * NC + lax.axis_index("c")
    row0 = wid * NCHUNK
    pltpu.sync_copy(text_hbm.at[pl.ds(row0, NCHUNK)], idx_v)

    sems = (sem0, sem1)

    def start(j, b):
        pltpu.async_copy(table_hbm.at[idx_v.at[j]], gbuf.at[b], sems[b])

    def wait(b):
        pltpu.make_async_copy(table_hbm.at[idx_v.at[0]], gbuf.at[b], sems[b]).wait()

    for b in range(NBUF):
        start(b, b)

    zeros = jnp.zeros((16,), jnp.float32)

    @pl.loop(0, NCHUNK, step=NBUF)
    def _chunks(g):
        for b in range(NBUF):
            j = g + b
            wait(b)

            for bag_in_chunk in range(BAGS_PER_CHUNK):
                r0 = bag_in_chunk * L

                def body(r, accs):
                    row = r0 + r
                    return tuple(
                        accs[d] + gbuf[b, row, pl.ds(d * 16, 16)]
                        for d in range(4)
                    )

                accs = lax.fori_loop(0, L, body, (zeros,) * 4, unroll=2)
                bag = j * BAGS_PER_CHUNK + bag_in_chunk
                for d in range(4):
                    outb[bag, pl.ds(d * 16, 16)] = accs[d]

            @pl.when(j + NBUF < NCHUNK)
            def _():
                start(j + NBUF, b)

    pltpu.sync_copy(outb, out_hbm.at[pl.ds(wid * BAGS_PER_W, BAGS_PER_W)])


@functools.partial(
    pl.kernel,
    out_type=jax.ShapeDtypeStruct((B, D), jnp.float32),
    mesh=plsc.VectorSubcoreMesh(core_axis_name="c", subcore_axis_name="s"),
    scratch_types=[
        pltpu.VMEM((NCHUNK, ROWS_PER_CHUNK), jnp.int32),
        pltpu.VMEM((NBUF, ROWS_PER_CHUNK, D), jnp.float32),
        pltpu.VMEM((BAGS_PER_W, D), jnp.float32),
        pltpu.SemaphoreType.DMA,
        pltpu.SemaphoreType.DMA,
    ],
)
def _sc_bag_sums(text_hbm, table_hbm, out_hbm, idx_v, gbuf, outb, sem0, sem1):
    _sc_body(text_hbm, table_hbm, out_hbm, idx_v, gbuf, outb, sem0, sem1)


def _tc_linear_body(x_ref, w_ref, b_ref, o_ref):
    o_ref[...] = (
        jnp.dot(x_ref[...], w_ref[...], preferred_element_type=jnp.float32)
        + b_ref[...]
    )


def _tc_linear(sums, w_pad, b_pad):
    blk = 2048
    return pl.pallas_call(
        _tc_linear_body,
        grid=(B // blk,),
        in_specs=[
            pl.BlockSpec((blk, D), lambda i: (i, 0)),
            pl.BlockSpec((D, 128), lambda i: (0, 0)),
            pl.BlockSpec((1, 128), lambda i: (0, 0)),
        ],
        out_specs=pl.BlockSpec((blk, 128), lambda i: (i, 0)),
        out_shape=jax.ShapeDtypeStruct((B, 128), jnp.float32),
    )(sums, w_pad, b_pad)


def kernel(text, emb_table, fc_w, fc_b):
    text2d = text.astype(jnp.int32).reshape(NW * NCHUNK, ROWS_PER_CHUNK)
    sums = _sc_bag_sums(text2d, emb_table)
    w_pad = jnp.zeros((D, 128), jnp.float32).at[:, :NUM_CLASS].set(fc_w.T / L)
    b_pad = jnp.zeros((1, 128), jnp.float32).at[0, :NUM_CLASS].set(fc_b)
    out = _tc_linear(sums, w_pad, b_pad)
    return out[:, :NUM_CLASS]


# speedup vs baseline: 2.5483x; 2.5483x over previous
import functools

import jax
import jax.numpy as jnp
from jax import lax
from jax.experimental import pallas as pl
from jax.experimental.pallas import tpu as pltpu
from jax.experimental.pallas import tpu_sc as plsc

VOCAB = 1000000
D = 64
B = 16384
L = 50
NUM_CLASS = 5

NC = 2
NS = 16
NW = NC * NS
BAGS_PER_W = B // NW
BAGS_PER_CHUNK = 2
ROWS_PER_CHUNK = BAGS_PER_CHUNK * L
NCHUNK = BAGS_PER_W // BAGS_PER_CHUNK
NBUF = 2


def _sc_body(text_hbm, table_hbm, out_hbm, idx_v, gbuf, outb, sem0, sem1):
    wid = lax.axis_index("s") * NC + lax.axis_index("c")
    row0 = wid * NCHUNK
    pltpu.sync_copy(text_hbm.at[pl.ds(row0, NCHUNK)], idx_v)

    sems = (sem0, sem1)

    def start(j, b):
        pltpu.async_copy(table_hbm.at[idx_v.at[j]], gbuf.at[b], sems[b])

    def wait(b):
        pltpu.make_async_copy(table_hbm.at[idx_v.at[0]], gbuf.at[b], sems[b]).wait()

    for b in range(NBUF):
        start(b, b)

    zeros = jnp.zeros((16,), jnp.float32)

    @pl.loop(0, NCHUNK, step=NBUF)
    def _chunks(g):
        for b in range(NBUF):
            j = g + b
            wait(b)

            for bag_in_chunk in range(BAGS_PER_CHUNK):
                r0 = bag_in_chunk * L

                def body(r, accs):
                    row = r0 + r
                    return tuple(
                        accs[d] + gbuf[b, row, pl.ds(d * 16, 16)]
                        for d in range(4)
                    )

                accs = lax.fori_loop(0, L, body, (zeros,) * 4, unroll=2)
                bag = j * BAGS_PER_CHUNK + bag_in_chunk
                for d in range(4):
                    outb[bag, pl.ds(d * 16, 16)] = accs[d]

            @pl.when(j + NBUF < NCHUNK)
            def _():
                start(j + NBUF, b)

    pltpu.sync_copy(outb, out_hbm.at[pl.ds(wid * BAGS_PER_W, BAGS_PER_W)])


@functools.partial(
    pl.kernel,
    out_type=jax.ShapeDtypeStruct((B, D), jnp.float32),
    mesh=plsc.VectorSubcoreMesh(core_axis_name="c", subcore_axis_name="s"),
    scratch_types=[
        pltpu.VMEM((NCHUNK, ROWS_PER_CHUNK), jnp.int32),
        pltpu.VMEM((NBUF, ROWS_PER_CHUNK, D), jnp.float32),
        pltpu.VMEM((BAGS_PER_W, D), jnp.float32),
        pltpu.SemaphoreType.DMA,
        pltpu.SemaphoreType.DMA,
    ],
    compiler_params=pltpu.CompilerParams(use_tc_tiling_on_sc=False),
)
def _sc_bag_sums(text_hbm, table_hbm, out_hbm, idx_v, gbuf, outb, sem0, sem1):
    _sc_body(text_hbm, table_hbm, out_hbm, idx_v, gbuf, outb, sem0, sem1)


def _tc_linear_body(x_ref, w_ref, b_ref, o_ref):
    o_ref[...] = (
        jnp.dot(x_ref[...], w_ref[...], preferred_element_type=jnp.float32)
        + b_ref[...]
    )


def _tc_linear(sums, w_pad, b_pad):
    blk = 2048
    return pl.pallas_call(
        _tc_linear_body,
        grid=(B // blk,),
        in_specs=[
            pl.BlockSpec((blk, D), lambda i: (i, 0)),
            pl.BlockSpec((D, 128), lambda i: (0, 0)),
            pl.BlockSpec((1, 128), lambda i: (0, 0)),
        ],
        out_specs=pl.BlockSpec((blk, 128), lambda i: (i, 0)),
        out_shape=jax.ShapeDtypeStruct((B, 128), jnp.float32),
    )(sums, w_pad, b_pad)


def kernel(text, emb_table, fc_w, fc_b):
    text2d = text.astype(jnp.int32).reshape(NW * NCHUNK, ROWS_PER_CHUNK)
    sums = _sc_bag_sums(text2d, emb_table)
    w_pad = jnp.zeros((D, 128), jnp.float32).at[:, :NUM_CLASS].set(fc_w.T / L)
    b_pad = jnp.zeros((1, 128), jnp.float32).at[0, :NUM_CLASS].set(fc_b)
    out = _tc_linear(sums, w_pad, b_pad)
    return out[:, :NUM_CLASS]
